# baseline (speedup 1.0000x reference)
import functools
import math

import jax
import jax.numpy as jnp
from jax import lax
from jax.experimental import pallas as pl
from jax.experimental.pallas import tpu as pltpu
from jax.experimental.pallas import tpu_sc as plsc

_PAD = 0
_SMOOTHING = 0.1
_CONF = 1.0 - _SMOOTHING

_BV = 2000
_KV = 12288
_RS = 48


def _tc_colsum_build(v, n):
    vt = v - _KV
    grid = -(-vt // _BV)
    lim = vt - (grid - 1) * _BV

    def body(x_ref, sum_ref, x0_ref):
        j = pl.program_id(0)
        xb = x_ref[...]

        @pl.when(j == 0)
        def _init():
            sum_ref[...] = jnp.zeros_like(sum_ref)
            x0_ref[...] = xb[0:1, :]

        @pl.when(j != grid - 1)
        def _full():
            sum_ref[...] += jnp.sum(xb, axis=0, keepdims=True)

        @pl.when(j == grid - 1)
        def _partial():
            rid = lax.broadcasted_iota(jnp.int32, xb.shape, 0)
            sum_ref[...] += jnp.sum(
                jnp.where(rid < lim, xb, 0.0), axis=0, keepdims=True)

    def run(xt):
        return pl.pallas_call(
            body,
            grid=(grid,),
            in_specs=[pl.BlockSpec((_BV, n), lambda j: (j, 0))],
            out_specs=[
                pl.BlockSpec((1, n), lambda j: (0, 0)),
                pl.BlockSpec((1, n), lambda j: (0, 0)),
            ],
            out_shape=[
                jax.ShapeDtypeStruct((1, n), jnp.float32),
                jax.ShapeDtypeStruct((1, n), jnp.float32),
            ],
        )(xt)

    return run


@functools.lru_cache(maxsize=None)
def _sc_gather_build(n, v):
    info = plsc.get_sparse_core_info()
    nc, ns, L = info.num_cores, info.num_subcores, info.num_lanes
    nw = nc * ns
    bpw = n // nw
    mesh = plsc.VectorSubcoreMesh(core_axis_name="c", subcore_axis_name="s")

    rpt = _KV // nw
    nslab = rpt // _RS

    @functools.partial(
        pl.kernel,
        mesh=mesh,
        compiler_params=pltpu.CompilerParams(use_tc_tiling_on_sc=True),
        out_type=[
            jax.ShapeDtypeStruct((nw, L), jnp.float32),
            jax.ShapeDtypeStruct((nw, L), jnp.float32),
            jax.ShapeDtypeStruct((nw, n), jnp.float32),
        ],
        scratch_types=[
            pltpu.VMEM((bpw,), jnp.int32),
            pltpu.VMEM((8, 128), jnp.float32),
            pltpu.VMEM((L,), jnp.float32),
            pltpu.VMEM((L,), jnp.float32),
            pltpu.VMEM((_RS, n), jnp.float32),
            pltpu.VMEM((n,), jnp.float32),
        ],
    )
    def sc_gather(xt_hbm, tgt_hbm, gsum_hbm, cnt_hbm, csum_hbm,
                  tgt_v, buf_v, g_v, c_v, slab_v, acc_v):
        wid = lax.axis_index("s") * nc + lax.axis_index("c")
        base = wid * bpw
        pltpu.sync_copy(tgt_hbm.at[pl.ds(base, bpw)], tgt_v)
        lane = lax.iota(jnp.int32, L)
        g = jnp.zeros((L,), jnp.float32)
        c = jnp.zeros((L,), jnp.float32)
        for j in range(bpw // L):
            t16 = tgt_v[pl.ds(j * L, L)]
            c = c + jnp.where(t16 != _PAD, 1.0, 0.0)
        for i in range(bpw):
            t16 = tgt_v[pl.ds((i // L) * L, L)]
            ti = t16[i % L]
            nabs = base + i
            rtile = pl.multiple_of((ti >> 3) << 3, 8)
            ctile = pl.multiple_of((nabs >> 7) << 7, 128)
            pltpu.sync_copy(
                xt_hbm.at[pl.ds(rtile, 8), pl.ds(ctile, 128)], buf_v)
            nloc = nabs - ctile
            cseg = (nloc >> 4) << 4
            rr = ti & 7
            hit = jnp.zeros((L,), jnp.float32)
            for r in range(8):
                seg = buf_v[r, pl.ds(cseg, L)]
                coef = (1 - jnp.minimum(jnp.abs(rr - r), 1)).astype(
                    jnp.float32)
                hit = hit + seg * coef
            sel = jnp.where(lane == (nloc & (L - 1)), hit, 0.0)
            validf = jnp.minimum(ti, 1).astype(jnp.float32)
            g = g + sel * validf
        g_v[...] = g
        c_v[...] = c
        pltpu.sync_copy(g_v, gsum_hbm.at[wid])
        pltpu.sync_copy(c_v, cnt_hbm.at[wid])

        rbase = (v - _KV) + wid * rpt
        for s in range(nslab):
            pltpu.sync_copy(
                xt_hbm.at[pl.ds(rbase + s * _RS, _RS)], slab_v)

            def slab_body(jj, carry, _s=s):
                col = pl.multiple_of(jj * L, L)
                vec = slab_v[0, pl.ds(col, L)]
                for r in range(1, _RS):
                    vec = vec + slab_v[r, pl.ds(col, L)]
                if _s == 0:
                    acc_v[pl.ds(col, L)] = vec
                else:
                    acc_v[pl.ds(col, L)] = acc_v[pl.ds(col, L)] + vec
                return carry

            lax.fori_loop(0, n // L, slab_body, jnp.int32(0))
        pltpu.sync_copy(acc_v, csum_hbm.at[wid])

    return sc_gather


def kernel(x, target):
    v = x.shape[-1]
    x2 = x.reshape(-1, v)
    n = x2.shape[0]
    t = target.reshape(-1)
    xt = jnp.swapaxes(x2, 0, 1)

    eps = _SMOOTHING / (v - 2)
    c_row = (v - 2) * eps * math.log(eps) + _CONF * math.log(_CONF)

    gpart, cpart, csum = _sc_gather_build(n, v)(xt, t)
    sums, x0row = _tc_colsum_build(v, n)(xt)

    wf = (t != _PAD).astype(jnp.float32)
    colsums = sums[0] + jnp.sum(csum, axis=0)
    tcsum = jnp.dot(colsums - x0row[0], wf)
    gsum = jnp.sum(gpart)
    nval = jnp.sum(cpart)
    return nval * c_row - eps * tcsum - (_CONF - eps) * gsum


# speedup vs baseline: 1.0167x; 1.0167x over previous
import functools
import math

import jax
import jax.numpy as jnp
from jax import lax
from jax.experimental import pallas as pl
from jax.experimental.pallas import tpu as pltpu
from jax.experimental.pallas import tpu_sc as plsc

_PAD = 0
_SMOOTHING = 0.1
_CONF = 1.0 - _SMOOTHING

_BV = 2000


def _tc_colsum_build(v, n):
    grid = v // _BV

    def body(x_ref, sum_ref, x0_ref):
        j = pl.program_id(0)
        xb = x_ref[...]

        @pl.when(j == 0)
        def _init():
            sum_ref[...] = jnp.zeros_like(sum_ref)
            x0_ref[...] = xb[0:1, :]

        sum_ref[...] += jnp.sum(xb, axis=0, keepdims=True)

    def run(xt):
        return pl.pallas_call(
            body,
            grid=(grid,),
            in_specs=[pl.BlockSpec((_BV, n), lambda j: (j, 0))],
            out_specs=[
                pl.BlockSpec((1, n), lambda j: (0, 0)),
                pl.BlockSpec((1, n), lambda j: (0, 0)),
            ],
            out_shape=[
                jax.ShapeDtypeStruct((1, n), jnp.float32),
                jax.ShapeDtypeStruct((1, n), jnp.float32),
            ],
        )(xt)

    return run


@functools.lru_cache(maxsize=None)
def _sc_gather_build(n, v):
    info = plsc.get_sparse_core_info()
    nc, ns, L = info.num_cores, info.num_subcores, info.num_lanes
    nw = nc * ns
    bpw = n // nw
    mesh = plsc.VectorSubcoreMesh(core_axis_name="c", subcore_axis_name="s")

    @functools.partial(
        pl.kernel,
        mesh=mesh,
        compiler_params=pltpu.CompilerParams(use_tc_tiling_on_sc=True),
        out_type=[
            jax.ShapeDtypeStruct((nw, L), jnp.float32),
            jax.ShapeDtypeStruct((nw, L), jnp.float32),
        ],
        scratch_types=[
            pltpu.VMEM((bpw,), jnp.int32),
            pltpu.VMEM((8, 128), jnp.float32),
            pltpu.VMEM((L,), jnp.float32),
            pltpu.VMEM((L,), jnp.float32),
        ],
    )
    def sc_gather(xt_hbm, tgt_hbm, gsum_hbm, cnt_hbm,
                  tgt_v, buf_v, g_v, c_v):
        wid = lax.axis_index("s") * nc + lax.axis_index("c")
        base = wid * bpw
        pltpu.sync_copy(tgt_hbm.at[pl.ds(base, bpw)], tgt_v)
        lane = lax.iota(jnp.int32, L)
        g = jnp.zeros((L,), jnp.float32)
        c = jnp.zeros((L,), jnp.float32)
        for j in range(bpw // L):
            t16 = tgt_v[pl.ds(j * L, L)]
            c = c + jnp.where(t16 != _PAD, 1.0, 0.0)
        for i in range(bpw):
            t16 = tgt_v[pl.ds((i // L) * L, L)]
            ti = t16[i % L]
            nabs = base + i
            rtile = pl.multiple_of((ti >> 3) << 3, 8)
            ctile = pl.multiple_of((nabs >> 7) << 7, 128)
            pltpu.sync_copy(
                xt_hbm.at[pl.ds(rtile, 8), pl.ds(ctile, 128)], buf_v)
            nloc = nabs - ctile
            cseg = (nloc >> 4) << 4
            rr = ti & 7
            hit = jnp.zeros((L,), jnp.float32)
            for r in range(8):
                seg = buf_v[r, pl.ds(cseg, L)]
                coef = (1 - jnp.minimum(jnp.abs(rr - r), 1)).astype(
                    jnp.float32)
                hit = hit + seg * coef
            sel = jnp.where(lane == (nloc & (L - 1)), hit, 0.0)
            validf = jnp.minimum(ti, 1).astype(jnp.float32)
            g = g + sel * validf
        g_v[...] = g
        c_v[...] = c
        pltpu.sync_copy(g_v, gsum_hbm.at[wid])
        pltpu.sync_copy(c_v, cnt_hbm.at[wid])

    return sc_gather


def kernel(x, target):
    v = x.shape[-1]
    x2 = x.reshape(-1, v)
    n = x2.shape[0]
    t = target.reshape(-1)
    xt = jnp.swapaxes(x2, 0, 1)

    eps = _SMOOTHING / (v - 2)
    c_row = (v - 2) * eps * math.log(eps) + _CONF * math.log(_CONF)

    gpart, cpart = _sc_gather_build(n, v)(xt, t)
    sums, x0row = _tc_colsum_build(v, n)(xt)

    wf = (t != _PAD).astype(jnp.float32)
    tcsum = jnp.dot(sums[0] - x0row[0], wf)
    gsum = jnp.sum(gpart)
    nval = jnp.sum(cpart)
    return nval * c_row - eps * tcsum - (_CONF - eps) * gsum
